# baseline (speedup 1.0000x reference)
import jax
import jax.numpy as jnp
from jax.experimental import pallas as pl
from jax.experimental.pallas import tpu as pltpu

_N = 4096
_K = 32768
_C = 4
_KT = 1024
_BETA = 0.25


def _vq_body(z_ref, cbt_ref, idx_ref, zq_ref, loss_ref):
    z = z_ref[...]
    zsq = jnp.sum(z * z, axis=1, keepdims=True)

    def sweep(t, carry):
        best_d, best_i = carry
        cbt = cbt_ref[:, pl.ds(t * _KT, _KT)]
        csq = jnp.sum(cbt * cbt, axis=0, keepdims=True)
        m2 = jax.lax.dot_general(
            z, cbt, (((1,), (0,)), ((), ())),
            preferred_element_type=jnp.float32)
        d = (zsq + csq) - 2.0 * m2
        dmin = jnp.min(d, axis=1, keepdims=True)
        lane = jax.lax.broadcasted_iota(jnp.int32, (_N, _KT), 1)
        gidx = lane + t * _KT
        timin = jnp.min(
            jnp.where(d == dmin, gidx, jnp.int32(_K)),
            axis=1, keepdims=True)
        upd = dmin < best_d
        return (jnp.where(upd, dmin, best_d), jnp.where(upd, timin, best_i))

    init = (jnp.full((_N, 1), jnp.inf, jnp.float32),
            jnp.zeros((_N, 1), jnp.int32))
    best_d, best_i = jax.lax.fori_loop(0, _K // _KT, sweep, init)
    idx_ref[...] = best_i

    def gather(t, zq):
        cbt = cbt_ref[:, pl.ds(t * _KT, _KT)]
        lane = jax.lax.broadcasted_iota(jnp.int32, (_N, _KT), 1)
        onehot = (lane + t * _KT == best_i).astype(jnp.float32)
        return zq + jax.lax.dot_general(
            onehot, cbt, (((1,), (1,)), ((), ())),
            precision=jax.lax.Precision.HIGHEST,
            preferred_element_type=jnp.float32)

    zq = jax.lax.fori_loop(0, _K // _KT, gather,
                           jnp.zeros((_N, _C), jnp.float32))
    e = zq - z
    zq_ref[...] = z + e
    m = jnp.mean(e * e)
    loss_ref[...] = jnp.broadcast_to(_BETA * m + m, (1, 1))


def _vq_call(z_flat, cbt):
    return pl.pallas_call(
        _vq_body,
        out_shape=(
            jax.ShapeDtypeStruct((_N, 1), jnp.int32),
            jax.ShapeDtypeStruct((_N, _C), jnp.float32),
            jax.ShapeDtypeStruct((1, 1), jnp.float32),
        ),
    )(z_flat, cbt)


def kernel(z, codebook):
    b, c, h, w = z.shape
    z_flat = jnp.transpose(z, (0, 2, 3, 1)).reshape(-1, c)
    cbt = codebook.T
    idx, zq_st, loss = _vq_call(z_flat, cbt)
    z_q_out = jnp.transpose(zq_st.reshape(b, h, w, c), (0, 3, 1, 2))
    indices = idx.reshape(b, h, w)
    return z_q_out, loss[0, 0], indices


# speedup vs baseline: 3.0775x; 3.0775x over previous
import jax
import jax.numpy as jnp
from jax.experimental import pallas as pl
from jax.experimental.pallas import tpu as pltpu

_N = 4096
_K = 32768
_C = 4
_KT = 512
_NT = _K // _KT
_BETA = 0.25


def _vq_body(z_ref, cbtm2_ref, cb3_ref, idx_ref, zq_ref, loss_ref):
    z = z_ref[...]
    zsq = jnp.sum(z * z, axis=1, keepdims=True)

    def tile_d(t):
        cm2 = cbtm2_ref[:, pl.ds(t * _KT, _KT)]
        csq = 0.25 * jnp.sum(cm2 * cm2, axis=0, keepdims=True)
        m2n = jax.lax.dot_general(
            z, cm2, (((1,), (0,)), ((), ())),
            preferred_element_type=jnp.float32)
        return (zsq + csq) + m2n

    best_d = tile_d(0)
    best_t = jnp.zeros((_N, _KT), jnp.int32)
    for t in range(1, _NT):
        d = tile_d(t)
        upd = d < best_d
        best_d = jnp.where(upd, d, best_d)
        best_t = jnp.where(upd, t, best_t)

    dmin = jnp.min(best_d, axis=1, keepdims=True)
    lane = jax.lax.broadcasted_iota(jnp.int32, (_N, _KT), 1)
    gidx = best_t * _KT + lane
    sel = jnp.where(best_d == dmin, gidx, jnp.int32(_K))
    best_i = jnp.min(sel, axis=1, keepdims=True)
    idx_ref[...] = best_i

    hi = best_i >> 7
    lo = best_i & 127
    oh_hi = (jax.lax.broadcasted_iota(jnp.int32, (_N, 256), 1)
             == hi).astype(jnp.float32)
    oh_lo = (jax.lax.broadcasted_iota(jnp.int32, (_N, 128), 1)
             == lo).astype(jnp.float32)
    cols = []
    for c in range(_C):
        xc = jax.lax.dot_general(
            oh_hi, cb3_ref[c], (((1,), (0,)), ((), ())),
            precision=jax.lax.Precision.HIGHEST,
            preferred_element_type=jnp.float32)
        cols.append(jnp.sum(xc * oh_lo, axis=1, keepdims=True))
    zq = jnp.concatenate(cols, axis=1)

    e = zq - z
    zq_ref[...] = z + e
    m = jnp.mean(e * e)
    loss_ref[...] = jnp.broadcast_to(_BETA * m + m, (1, 1))


def _vq_call(z_flat, cbt_m2, cb3):
    return pl.pallas_call(
        _vq_body,
        out_shape=(
            jax.ShapeDtypeStruct((_N, 1), jnp.int32),
            jax.ShapeDtypeStruct((_N, _C), jnp.float32),
            jax.ShapeDtypeStruct((1, 1), jnp.float32),
        ),
    )(z_flat, cbt_m2, cb3)


def kernel(z, codebook):
    b, c, h, w = z.shape
    z_flat = jnp.transpose(z, (0, 2, 3, 1)).reshape(-1, c)
    cbt_m2 = codebook.T * jnp.float32(-2.0)
    cb3 = codebook.T.reshape(c, _K // 128, 128)
    idx, zq_st, loss = _vq_call(z_flat, cbt_m2, cb3)
    z_q_out = jnp.transpose(zq_st.reshape(b, h, w, c), (0, 3, 1, 2))
    indices = idx.reshape(b, h, w)
    return z_q_out, loss[0, 0], indices
